# baseline (speedup 1.0000x reference)
import jax
import jax.numpy as jnp
from jax import lax
from jax.experimental import pallas as pl
from jax.experimental.pallas import tpu as pltpu
from jax.experimental.pallas import tpu_sc as plsc

F = 26
V = 100000
D = 32
B = 16384

_info = plsc.get_sparse_core_info()
NC = _info.num_cores
NSUB = _info.num_subcores
L = _info.num_lanes
NW = NC * NSUB
IC = 4096
NCHK = B // IC
UNROLL = 8


def _body(idx_hbm, tab_hbm, out_hbm,
          rowbuf, idx0, idx1, acc, sem_i0, sem_i1):
    c = lax.axis_index("c")
    s = lax.axis_index("s")
    d = c * NSUB + s

    ibufs = (idx0, idx1)
    isems = (sem_i0, sem_i1)

    @plsc.parallel_loop(0, B, step=L)
    def _zero(b):
        acc[pl.ds(b, L)] = jnp.zeros((L,), jnp.float32)

    pltpu.async_copy(idx_hbm.at[0, pl.ds(0, IC)], ibufs[0], isems[0])

    def _field(i, _f):
        pltpu.sync_copy(tab_hbm.at[i, d], rowbuf)
        inext = jnp.minimum(i + 1, F - 1)
        for ch in range(NCHK):
            p = ch % 2
            q = (ch + 1) % 2
            ib = ibufs[p]
            pltpu.make_async_copy(idx_hbm.at[0, pl.ds(0, IC)], ib, isems[p]).wait()
            if ch + 1 < NCHK:
                pltpu.async_copy(idx_hbm.at[i, pl.ds((ch + 1) * IC, IC)],
                                 ibufs[q], isems[q])
            else:
                @pl.when(i + 1 < F)
                def _prefetch_next_field():
                    pltpu.async_copy(idx_hbm.at[inext, pl.ds(0, IC)],
                                     ibufs[q], isems[q])
            base = ch * IC

            @plsc.parallel_loop(0, IC, step=L, unroll=UNROLL)
            def _gather(b):
                v = ib[pl.ds(b, L)]
                vals = plsc.load_gather(rowbuf, [v])
                plsc.addupdate(acc.at[pl.ds(base + b, L)], vals)

        return 0

    lax.fori_loop(0, F, _field, 0)
    pltpu.sync_copy(acc, out_hbm.at[d])


def kernel(inputs, tables):
    idx_t = jnp.transpose(inputs).astype(jnp.int32)
    tab_t = jnp.transpose(tables, (0, 2, 1))
    mesh = plsc.VectorSubcoreMesh(core_axis_name="c", subcore_axis_name="s")
    f = pl.kernel(
        _body,
        out_type=jax.ShapeDtypeStruct((D, B), jnp.float32),
        mesh=mesh,
        scratch_types=[
            pltpu.VMEM((V,), jnp.float32),
            pltpu.VMEM((IC,), jnp.int32),
            pltpu.VMEM((IC,), jnp.int32),
            pltpu.VMEM((B,), jnp.float32),
            pltpu.SemaphoreType.DMA,
            pltpu.SemaphoreType.DMA,
        ],
        compiler_params=pltpu.CompilerParams(needs_layout_passes=False),
    )
    out_t = f(idx_t, tab_t)
    return jnp.transpose(out_t)
